# Initial kernel scaffold; baseline (speedup 1.0000x reference)
import jax
import jax.numpy as jnp
from jax.experimental import pallas as pl


def kernel(encoding, mention_batch_positions, mention_start_positions, mention_end_positions, mention_mask, memory_keys, memory_values, memory_mask, memory_entity_ids, w_query, b_query, w_value, b_value, ln_scale, ln_bias, deterministic):
    raise NotImplementedError("write your pallas kernel here")


# speedup vs baseline: 1.0098x; 1.0098x over previous
import functools

import jax
import jax.numpy as jnp
from jax import lax
from jax.experimental import pallas as pl
from jax.experimental.pallas import tpu as pltpu

K_TOP = 32
ROWS = 128
LN_EPS = 1e-12
_LARGE_NUMBER = 10000000000.0


def _queries_body(qin_ref, w_ref, b_ref, out_ref):
    out_ref[...] = (
        jnp.dot(qin_ref[...], w_ref[...], preferred_element_type=jnp.float32)
        + b_ref[...]
    )


def _compute_queries(qin, w_query, b_query):
    M = qin.shape[0]
    KD = w_query.shape[1]
    return pl.pallas_call(
        _queries_body,
        out_shape=jax.ShapeDtypeStruct((M, KD), jnp.float32),
    )(qin, w_query, b_query.reshape(1, KD))


def _scores_body(q_ref, k_ref, s_ref, rmax_ref, *, rows_per_blk, cols):
    s = jnp.dot(
        q_ref[...], k_ref[...].T, preferred_element_type=jnp.float32
    )
    s_ref[...] = s
    m = s.reshape(s.shape[0], rows_per_blk, cols).max(axis=-1)
    rmax_ref[...] = m[None]


def _compute_scores(queries, memory_keys):
    M, KD = queries.shape
    MEM = memory_keys.shape[0]
    cols = MEM // ROWS
    BLK = 2048
    rows_per_blk = BLK // cols
    grid = MEM // BLK
    return pl.pallas_call(
        functools.partial(_scores_body, rows_per_blk=rows_per_blk, cols=cols),
        grid=(grid,),
        in_specs=[
            pl.BlockSpec((M, KD), lambda i: (0, 0)),
            pl.BlockSpec((BLK, KD), lambda i: (i, 0)),
        ],
        out_specs=[
            pl.BlockSpec((M, BLK), lambda i: (0, i)),
            pl.BlockSpec((1, M, rows_per_blk), lambda i: (i, 0, 0)),
        ],
        out_shape=[
            jax.ShapeDtypeStruct((M, MEM), jnp.float32),
            jax.ShapeDtypeStruct((grid, M, rows_per_blk), jnp.float32),
        ],
    )(queries, memory_keys)


def _finish_body(enc_ref, upd_ref, pos_ref, scale_ref, bias_ref, out_ref, *, tpb):
    i = pl.program_id(0)
    pos = pos_ref[...].reshape(-1)
    tok = lax.broadcasted_iota(jnp.int32, (pos.shape[0], tpb), 1) + i * tpb
    oh = (pos[:, None] == tok).astype(jnp.float32)
    scat = lax.dot_general(
        oh, upd_ref[...], (((0,), (0,)), ((), ())),
        preferred_element_type=jnp.float32,
    )
    enc = enc_ref[...] + scat
    mean = jnp.mean(enc, axis=-1, keepdims=True)
    var = jnp.mean((enc - mean) ** 2, axis=-1, keepdims=True)
    out_ref[...] = (enc - mean) * lax.rsqrt(var + LN_EPS) * scale_ref[...] + bias_ref[...]


def _finish(encoding_flat, update, pos, ln_scale, ln_bias):
    N, H = encoding_flat.shape
    M = update.shape[0]
    TPB = 256
    grid = N // TPB
    return pl.pallas_call(
        functools.partial(_finish_body, tpb=TPB),
        grid=(grid,),
        in_specs=[
            pl.BlockSpec((TPB, H), lambda i: (i, 0)),
            pl.BlockSpec((M, H), lambda i: (0, 0)),
            pl.BlockSpec((1, M), lambda i: (0, 0)),
            pl.BlockSpec((1, H), lambda i: (0, 0)),
            pl.BlockSpec((1, H), lambda i: (0, 0)),
        ],
        out_specs=pl.BlockSpec((TPB, H), lambda i: (i, 0)),
        out_shape=jax.ShapeDtypeStruct((N, H), jnp.float32),
    )(encoding_flat, update, pos.reshape(1, M), ln_scale.reshape(1, H),
      ln_bias.reshape(1, H))


def _update_body(ret_ref, w_ref, b_ref, mask_ref, out_ref):
    upd = (
        jnp.dot(ret_ref[...], w_ref[...], preferred_element_type=jnp.float32)
        + b_ref[...]
    )
    out_ref[...] = upd * mask_ref[...].reshape(-1, 1).astype(jnp.float32)


def _compute_update(retrieved, w_value, b_value, mention_mask):
    M = retrieved.shape[0]
    H = w_value.shape[1]
    return pl.pallas_call(
        _update_body,
        out_shape=jax.ShapeDtypeStruct((M, H), jnp.float32),
    )(retrieved, w_value, b_value.reshape(1, H), mention_mask.reshape(M, 1))


def kernel(encoding, mention_batch_positions, mention_start_positions,
           mention_end_positions, mention_mask, memory_keys, memory_values,
           memory_mask, memory_entity_ids, w_query, b_query, w_value,
           b_value, ln_scale, ln_bias, deterministic=True):
    B, T, H = encoding.shape
    M = mention_batch_positions.shape[0]
    MEM, KD = memory_keys.shape
    flat = encoding.reshape(B * T, H)

    start_pos = (mention_batch_positions * T + mention_start_positions).astype(jnp.int32)
    end_pos = (mention_batch_positions * T + mention_end_positions).astype(jnp.int32)
    qin = jnp.concatenate((flat[start_pos], flat[end_pos]), axis=-1)
    queries = _compute_queries(qin, w_query, b_query)

    scores, rowmax3 = _compute_scores(queries, memory_keys)
    rowmax = jnp.transpose(rowmax3, (1, 0, 2)).reshape(M, ROWS)

    cols = MEM // ROWS
    s3 = scores.reshape(M, ROWS, cols)
    row_scores, row_ids = lax.top_k(s3, K_TOP)
    global_ids = row_ids + (jnp.arange(ROWS) * cols)[None, :, None]
    cand_scores = row_scores.reshape(M, ROWS * K_TOP)
    cand_ids = global_ids.reshape(M, ROWS * K_TOP)
    top_scores, top_idx = lax.top_k(cand_scores, K_TOP)
    top_ids = jnp.take_along_axis(cand_ids, top_idx, axis=1)

    top_entity_ids = memory_entity_ids[top_ids]
    top_mask = memory_mask[top_ids].astype(jnp.float32)
    logits = top_scores - (1.0 - top_mask) * _LARGE_NUMBER
    attention_weights = jax.nn.softmax(logits, axis=-1)
    top_values = memory_values[top_ids]
    retrieved = jnp.einsum('qk,qkd->qd', attention_weights, top_values)

    update = _compute_update(retrieved, w_value, b_value, mention_mask)
    normed = _finish(flat, update, start_pos, ln_scale,
                     ln_bias).reshape(B, T, H)
    return (normed, attention_weights, top_entity_ids)


# speedup vs baseline: 11.3542x; 11.2442x over previous
import functools

import jax
import jax.numpy as jnp
from jax import lax
from jax.experimental import pallas as pl
from jax.experimental.pallas import tpu as pltpu
from jax.experimental.pallas import tpu_sc as plsc

K_TOP = 32
ROWS = 128
LN_EPS = 1e-12
_LARGE_NUMBER = 10000000000.0

_NC, _NS, _L = 2, 16, 16
_NW = _NC * _NS
_K_ROWS = 48
_CAND = 64
_NEG = -3.0e38


def _queries_body(qin_ref, w_ref, b_ref, out_ref):
    out_ref[...] = (
        jnp.dot(qin_ref[...], w_ref[...], preferred_element_type=jnp.float32)
        + b_ref[...]
    )


def _compute_queries(qin, w_query, b_query):
    M = qin.shape[0]
    KD = w_query.shape[1]
    return pl.pallas_call(
        _queries_body,
        out_shape=jax.ShapeDtypeStruct((M, KD), jnp.float32),
    )(qin, w_query, b_query.reshape(1, KD))


def _scores_body(q_ref, k_ref, s_ref, rmax_ref, *, rows_per_blk, cols):
    s = jnp.dot(
        q_ref[...], k_ref[...].T, preferred_element_type=jnp.float32
    )
    s_ref[...] = s
    m = s.reshape(s.shape[0], rows_per_blk, cols).max(axis=-1)
    rmax_ref[...] = m[None]


def _compute_scores(queries, memory_keys):
    M, KD = queries.shape
    MEM = memory_keys.shape[0]
    cols = MEM // ROWS
    BLK = 2048
    rows_per_blk = BLK // cols
    grid = MEM // BLK
    return pl.pallas_call(
        functools.partial(_scores_body, rows_per_blk=rows_per_blk, cols=cols),
        grid=(grid,),
        in_specs=[
            pl.BlockSpec((M, KD), lambda i: (0, 0)),
            pl.BlockSpec((BLK, KD), lambda i: (i, 0)),
        ],
        out_specs=[
            pl.BlockSpec((M, BLK), lambda i: (0, i)),
            pl.BlockSpec((1, M, rows_per_blk), lambda i: (i, 0, 0)),
        ],
        out_shape=[
            jax.ShapeDtypeStruct((M, MEM), jnp.float32),
            jax.ShapeDtypeStruct((grid, M, rows_per_blk), jnp.float32),
        ],
    )(queries, memory_keys)


def _s(v):
    return lax.squeeze(lax.slice_in_dim(v, 0, 1), (0,))


def _iota16():
    return lax.iota(jnp.int32, 16)


def _perm(v, idx):
    return v.at[idx].get(mode="promise_in_bounds")


def _allmax(v):
    idx = _iota16()
    for d in (1, 2, 4, 8):
        v = jnp.maximum(v, _perm(v, idx ^ d))
    return v


def _allsum(v):
    idx = _iota16()
    for d in (1, 2, 4, 8):
        v = v + _perm(v, idx ^ d)
    return v


def _mi(m):
    return jnp.where(m, jnp.full((16,), 1, jnp.int32),
                     jnp.zeros((16,), jnp.int32))


def _prefix_sum_incl(x):
    idx = _iota16()
    for d in (1, 2, 4, 8):
        sh = _perm(x, jnp.maximum(idx - d, 0))
        x = x + jnp.where(idx >= d, sh, jnp.zeros((16,), x.dtype))
    return x


def _cmpex(k, v, j, wm):
    idx = _iota16()
    pidx = idx ^ j
    pk = _perm(k, pidx)
    pv = _perm(v, pidx)
    mx = jnp.maximum(k, pk)
    mn = jnp.minimum(k, pk)
    nk = jnp.where(wm > 0, mx, mn)
    took = nk != k
    return nk, jnp.where(took, pv, v)


def _log2(j):
    return j.bit_length() - 1


def _sort16_desc(k, v):
    idx = _iota16()
    for size in (2, 4, 8, 16):
        bit = lax.shift_right_logical(idx, _log2(size)) & 1
        j = size // 2
        while j >= 1:
            upper = lax.shift_right_logical(idx, _log2(j)) & 1
            k, v = _cmpex(k, v, j, (1 - bit) ^ upper)
            j //= 2
    return k, v


def _bmerge16_desc(k, v):
    idx = _iota16()
    for j in (8, 4, 2, 1):
        upper = lax.shift_right_logical(idx, _log2(j)) & 1
        k, v = _cmpex(k, v, j, 1 - upper)
    return k, v


def _merge16_desc(ak, av, bk, bv):
    rbk = jnp.flip(bk, 0)
    rbv = jnp.flip(bv, 0)
    m = ak >= rbk
    hk = jnp.where(m, ak, rbk)
    hv = jnp.where(m, av, rbv)
    lk = jnp.where(m, rbk, ak)
    lv = jnp.where(m, rbv, av)
    hk, hv = _bmerge16_desc(hk, hv)
    lk, lv = _bmerge16_desc(lk, lv)
    return hk, hv, lk, lv


def _cross(ak, av, bk, bv):
    m = ak >= bk
    return (jnp.where(m, ak, bk), jnp.where(m, av, bv),
            jnp.where(m, bk, ak), jnp.where(m, bv, av))


def _top32_of_64_desc(keys, vals):
    s = [_sort16_desc(keys[j], vals[j]) for j in range(4)]
    a0k, a0v, a1k, a1v = _merge16_desc(s[0][0], s[0][1], s[1][0], s[1][1])
    b0k, b0v, b1k, b1v = _merge16_desc(s[2][0], s[2][1], s[3][0], s[3][1])
    rb0k, rb0v = jnp.flip(b1k, 0), jnp.flip(b1v, 0)
    rb1k, rb1v = jnp.flip(b0k, 0), jnp.flip(b0v, 0)
    h0k, h0v, _, _ = _cross(a0k, a0v, rb0k, rb0v)
    h1k, h1v, _, _ = _cross(a1k, a1v, rb1k, rb1v)
    h0k, h0v, h1k, h1v = _cross(h0k, h0v, h1k, h1v)
    h0k, h0v = _bmerge16_desc(h0k, h0v)
    h1k, h1v = _bmerge16_desc(h1k, h1v)
    return (h0k, h1k), (h0v, h1v)


def _sc_select_attend(scores3d, rowmax, memory_values, eids, mmask):
    M = rowmax.shape[0]
    COLS = scores3d.shape[1] * scores3d.shape[2]
    VD = memory_values.shape[1]
    QPW = M // _NW
    mesh = plsc.VectorSubcoreMesh(core_axis_name="c", subcore_axis_name="s")

    @functools.partial(
        pl.kernel,
        mesh=mesh,
        out_type=[
            jax.ShapeDtypeStruct((M, K_TOP), jnp.float32),
            jax.ShapeDtypeStruct((M, K_TOP), jnp.int32),
            jax.ShapeDtypeStruct((M, VD), jnp.float32),
        ],
        scratch_types=[
            pltpu.VMEM((ROWS,), jnp.float32),
            pltpu.VMEM((64,), jnp.int32),
            pltpu.VMEM((_K_ROWS, 8, COLS // 8), jnp.float32),
            pltpu.VMEM((_CAND,), jnp.float32),
            pltpu.VMEM((_CAND,), jnp.int32),
            pltpu.VMEM((K_TOP,), jnp.int32),
            pltpu.VMEM((K_TOP, VD), jnp.float32),
            pltpu.VMEM((K_TOP,), jnp.int32),
            pltpu.VMEM((K_TOP,), jnp.int32),
            pltpu.VMEM((K_TOP,), jnp.float32),
            pltpu.VMEM((VD,), jnp.float32),
            pltpu.VMEM((16,), jnp.int32),
            pltpu.SemaphoreType.DMA,
            pltpu.SemaphoreType.DMA,
            pltpu.SemaphoreType.DMA,
        ],
    )
    def k(scores_hbm, rowmax_hbm, values_hbm, eids_hbm, mmask_hbm,
          attw_hbm, eout_hbm, ret_hbm,
          rm_v, rid_v, grows_v, cv_v, ci_v, tid_v, vals_v, eid_v,
          msk_v, w_v, ret_v, cb_v, sem0, sem1, sem2):
        wid = lax.axis_index("s") * _NC + lax.axis_index("c")

        for j in range(4):
            rid_v[pl.ds(16 * j, 16)] = jnp.zeros((16,), jnp.int32)

        def per_query(it, _):
            q = wid * QPW + it

            pltpu.sync_copy(rowmax_hbm.at[q], rm_v)
            rms = [rm_v[pl.ds(16 * j, 16)] for j in range(ROWS // 16)]
            mn = rms[0]
            mx = rms[0]
            for v in rms[1:]:
                mn = jnp.minimum(mn, v)
                mx = jnp.maximum(mx, v)
            lo = -_s(_allmax(-mn))
            hi = _s(_allmax(mx))
            for _bs in range(28):
                mid = 0.5 * (lo + hi)
                cnt = jnp.zeros((16,), jnp.int32)
                for v in rms:
                    cnt = cnt + _mi(v >= mid)
                ge = _s(_allsum(cnt)) >= K_TOP
                lo = jnp.where(ge, mid, lo)
                hi = jnp.where(ge, hi, mid)
            t = lo

            base = jnp.zeros((), jnp.int32)
            for j in range(ROWS // 16):
                m = rms[j] >= t
                mi = _mi(m)
                pfx = _prefix_sum_incl(mi) - mi
                key = jnp.where(m, 1000 - pfx, jnp.full((16,), -1, jnp.int32))
                _, sperm = _sort16_desc(key, _iota16())
                gid = q * ROWS + 16 * j + sperm
                rid_v[pl.ds(base, 16)] = gid
                base = jnp.minimum(base + _s(_allsum(mi)),
                                   jnp.full((), _K_ROWS, jnp.int32))
            count = base

            pltpu.async_copy(
                scores_hbm.at[rid_v.at[pl.ds(0, _K_ROWS)]], grows_v, sem0
            ).wait()

            for j in range(_CAND // 16):
                cv_v[pl.ds(16 * j, 16)] = jnp.full((16,), _NEG, jnp.float32)
                ci_v[pl.ds(16 * j, 16)] = jnp.zeros((16,), jnp.int32)
            cb_v[...] = jnp.zeros((16,), jnp.int32)

            idx = _iota16()

            def per_row(r, _):
                rvalid = r < count
                for c0 in range(COLS // 128):
                    vs = [grows_v[r, c0, pl.ds(16 * u, 16)]
                          for u in range(8)]
                    gmax = vs[0]
                    for u in range(1, 8):
                        gmax = jnp.maximum(gmax, vs[u])
                    vmax = _s(_allmax(gmax))
                    hit = (vmax >= t) & rvalid

                    def detail():
                        cntv = jnp.zeros((16,), jnp.int32)
                        for u in range(8):
                            cntv = cntv + _mi(vs[u] >= t)
                        cnt = _s(_allsum(cntv))
                        vcur = list(vs)
                        vmax0 = vmax
                        for rnd in range(3):
                            if rnd == 0:
                                vm = vmax0
                            else:
                                gm = vcur[0]
                                for u in range(1, 8):
                                    gm = jnp.maximum(gm, vcur[u])
                                vm = _s(_allmax(gm))
                            posv = jnp.full((16,), 999, jnp.int32)
                            for u in range(8):
                                posv = jnp.minimum(
                                    posv,
                                    jnp.where(vcur[u] == vm, 16 * u + idx,
                                              jnp.full((16,), 999, jnp.int32)))
                            pos = -_s(_allmax(-posv))
                            enc1 = r * COLS + c0 * 128 + pos
                            cb = _s(cb_v[...])
                            cv_v[pl.ds(cb, 16)] = jnp.where(
                                idx == 0, jnp.full((16,), vm, jnp.float32),
                                jnp.full((16,), _NEG, jnp.float32))
                            ci_v[pl.ds(cb, 16)] = jnp.where(
                                idx == 0, jnp.full((16,), enc1, jnp.int32),
                                jnp.zeros((16,), jnp.int32))
                            cb_v[...] = jnp.minimum(
                                cb_v[...] + jnp.where(cnt > rnd, 1, 0),
                                jnp.full((16,), _CAND - 16, jnp.int32))
                            if rnd < 2:
                                vnext = []
                                for u in range(8):
                                    vnext.append(jnp.where(
                                        (16 * u + idx) == pos,
                                        jnp.full((16,), _NEG, jnp.float32),
                                        vcur[u]))
                                vcur = vnext

                    lax.cond(hit, detail, lambda: None)
                return 0

            lax.fori_loop(0, _K_ROWS, per_row, 0)


            ck = [cv_v[pl.ds(16 * j, 16)] for j in range(4)]
            cid = [ci_v[pl.ds(16 * j, 16)] for j in range(4)]
            (tk0, tk1), (tid0, tid1) = _top32_of_64_desc(ck, cid)

            rid0 = rid_v[pl.ds(0, 16)]
            rid1 = rid_v[pl.ds(16, 16)]
            rid2 = rid_v[pl.ds(32, 16)]

            def remap(enc):
                rslot = lax.shift_right_logical(enc, 10)
                col = enc & (COLS - 1)
                low = rslot & 15
                g0 = _perm(rid0, low)
                g1 = _perm(rid1, low)
                g2 = _perm(rid2, low)
                gro = jnp.where(rslot < 16, g0, jnp.where(rslot < 32, g1, g2))
                return (gro - q * ROWS) * COLS + col

            tid_v[pl.ds(0, 16)] = remap(tid0)
            tid_v[pl.ds(16, 16)] = remap(tid1)

            cpv = pltpu.async_copy(values_hbm.at[tid_v], vals_v, sem0)
            cpe = pltpu.async_copy(eids_hbm.at[tid_v], eid_v, sem1)
            cpm = pltpu.async_copy(mmask_hbm.at[tid_v], msk_v, sem2)
            cpv.wait()
            cpe.wait()
            cpm.wait()
            msk0 = msk_v[pl.ds(0, 16)]
            msk1 = msk_v[pl.ds(16, 16)]

            pen0 = jnp.where(msk0 > 0, jnp.zeros((16,), jnp.float32),
                             jnp.full((16,), _LARGE_NUMBER, jnp.float32))
            pen1 = jnp.where(msk1 > 0, jnp.zeros((16,), jnp.float32),
                             jnp.full((16,), _LARGE_NUMBER, jnp.float32))
            l0 = tk0 - pen0
            l1 = tk1 - pen1
            mxl = _s(_allmax(jnp.maximum(l0, l1)))
            e0 = jnp.exp(l0 - mxl)
            e1 = jnp.exp(l1 - mxl)
            ssum = _s(_allsum(e0 + e1))
            w0 = e0 / ssum
            w1 = e1 / ssum
            w_v[pl.ds(0, 16)] = w0
            w_v[pl.ds(16, 16)] = w1

            wks = [_perm(w0 if kk < 16 else w1,
                         jnp.full((16,), kk % 16, jnp.int32))
                   for kk in range(K_TOP)]

            def per_chunk(jc, _):
                acc = jnp.zeros((16,), jnp.float32)
                for kk in range(K_TOP):
                    acc = acc + wks[kk] * vals_v[kk, pl.ds(jc * 16, 16)]
                ret_v[pl.ds(jc * 16, 16)] = acc
                return 0

            lax.fori_loop(0, VD // 16, per_chunk, 0)

            pltpu.sync_copy(w_v, attw_hbm.at[q])
            pltpu.sync_copy(eid_v, eout_hbm.at[q])
            pltpu.sync_copy(ret_v, ret_hbm.at[q])
            return 0

        lax.fori_loop(0, QPW, per_query, 0)

    return k(scores3d, rowmax, memory_values, eids, mmask)


def _finish_body(enc_ref, ret_ref, wv_ref, bv_ref, mm_ref, pos_ref,
                 scale_ref, bias_ref, out_ref, *, tpb):
    i = pl.program_id(0)
    upd = (
        jnp.dot(ret_ref[...], wv_ref[...], preferred_element_type=jnp.float32)
        + bv_ref[...]
    ) * mm_ref[...].reshape(-1, 1).astype(jnp.float32)
    pos = pos_ref[...].reshape(-1)
    tok = lax.broadcasted_iota(jnp.int32, (pos.shape[0], tpb), 1) + i * tpb
    oh = (pos[:, None] == tok).astype(jnp.float32)
    scat = lax.dot_general(
        oh, upd, (((0,), (0,)), ((), ())),
        preferred_element_type=jnp.float32,
    )
    enc = enc_ref[...] + scat
    mean = jnp.mean(enc, axis=-1, keepdims=True)
    var = jnp.mean((enc - mean) ** 2, axis=-1, keepdims=True)
    out_ref[...] = (enc - mean) * lax.rsqrt(var + LN_EPS) * scale_ref[...] + bias_ref[...]


def _finish(encoding_flat, retrieved, w_value, b_value, mention_mask, pos,
            ln_scale, ln_bias):
    N, H = encoding_flat.shape
    M, VD = retrieved.shape
    TPB = 256
    grid = N // TPB
    return pl.pallas_call(
        functools.partial(_finish_body, tpb=TPB),
        grid=(grid,),
        in_specs=[
            pl.BlockSpec((TPB, H), lambda i: (i, 0)),
            pl.BlockSpec((M, VD), lambda i: (0, 0)),
            pl.BlockSpec((VD, H), lambda i: (0, 0)),
            pl.BlockSpec((1, H), lambda i: (0, 0)),
            pl.BlockSpec((M, 1), lambda i: (0, 0)),
            pl.BlockSpec((1, M), lambda i: (0, 0)),
            pl.BlockSpec((1, H), lambda i: (0, 0)),
            pl.BlockSpec((1, H), lambda i: (0, 0)),
        ],
        out_specs=pl.BlockSpec((TPB, H), lambda i: (i, 0)),
        out_shape=jax.ShapeDtypeStruct((N, H), jnp.float32),
    )(encoding_flat, retrieved, w_value, b_value.reshape(1, H),
      mention_mask.reshape(M, 1), pos.reshape(1, M), ln_scale.reshape(1, H),
      ln_bias.reshape(1, H))


def kernel(encoding, mention_batch_positions, mention_start_positions,
           mention_end_positions, mention_mask, memory_keys, memory_values,
           memory_mask, memory_entity_ids, w_query, b_query, w_value,
           b_value, ln_scale, ln_bias, deterministic=True):
    B, T, H = encoding.shape
    M = mention_batch_positions.shape[0]
    MEM, KD = memory_keys.shape
    cols = MEM // ROWS
    flat = encoding.reshape(B * T, H)

    start_pos = (mention_batch_positions * T + mention_start_positions).astype(jnp.int32)
    end_pos = (mention_batch_positions * T + mention_end_positions).astype(jnp.int32)
    qin = jnp.concatenate((flat[start_pos], flat[end_pos]), axis=-1)
    queries = _compute_queries(qin, w_query, b_query)

    scores, rowmax3 = _compute_scores(queries, memory_keys)
    rowmax = jnp.transpose(rowmax3, (1, 0, 2)).reshape(M, ROWS)

    attention_weights, top_entity_ids, retrieved = _sc_select_attend(
        scores.reshape(M * ROWS, 8, cols // 8), rowmax, memory_values,
        memory_entity_ids.astype(jnp.int32), memory_mask.astype(jnp.int32))

    normed = _finish(flat, retrieved, w_value, b_value, mention_mask,
                     start_pos, ln_scale, ln_bias).reshape(B, T, H)
    return (normed, attention_weights, top_entity_ids)


# speedup vs baseline: 11.9954x; 1.0565x over previous
import functools

import jax
import jax.numpy as jnp
from jax import lax
from jax.experimental import pallas as pl
from jax.experimental.pallas import tpu as pltpu
from jax.experimental.pallas import tpu_sc as plsc

K_TOP = 32
ROWS = 128
LN_EPS = 1e-12
_LARGE_NUMBER = 10000000000.0

_NC, _NS, _L = 2, 16, 16
_NW = _NC * _NS
_K_ROWS = 40
_CAND = 64
_NEG = -3.0e38


def _queries_body(qin_ref, w_ref, b_ref, out_ref):
    out_ref[...] = (
        jnp.dot(qin_ref[...], w_ref[...], preferred_element_type=jnp.float32)
        + b_ref[...]
    )


def _compute_queries(qin, w_query, b_query):
    M = qin.shape[0]
    KD = w_query.shape[1]
    return pl.pallas_call(
        _queries_body,
        out_shape=jax.ShapeDtypeStruct((M, KD), jnp.float32),
    )(qin, w_query, b_query.reshape(1, KD))


def _scores_body(q_ref, k_ref, s_ref, rmax_ref, *, rows_per_blk, cols):
    s = jnp.dot(
        q_ref[...], k_ref[...].T, preferred_element_type=jnp.float32
    )
    s_ref[...] = s
    m = s.reshape(s.shape[0], rows_per_blk, cols).max(axis=-1)
    rmax_ref[...] = m[None]


def _compute_scores(queries, memory_keys):
    M, KD = queries.shape
    MEM = memory_keys.shape[0]
    cols = MEM // ROWS
    BLK = 2048
    rows_per_blk = BLK // cols
    grid = MEM // BLK
    return pl.pallas_call(
        functools.partial(_scores_body, rows_per_blk=rows_per_blk, cols=cols),
        grid=(grid,),
        in_specs=[
            pl.BlockSpec((M, KD), lambda i: (0, 0)),
            pl.BlockSpec((BLK, KD), lambda i: (i, 0)),
        ],
        out_specs=[
            pl.BlockSpec((M, BLK), lambda i: (0, i)),
            pl.BlockSpec((1, M, rows_per_blk), lambda i: (i, 0, 0)),
        ],
        out_shape=[
            jax.ShapeDtypeStruct((M, MEM), jnp.float32),
            jax.ShapeDtypeStruct((grid, M, rows_per_blk), jnp.float32),
        ],
    )(queries, memory_keys)


def _s(v):
    return lax.squeeze(lax.slice_in_dim(v, 0, 1), (0,))


def _iota16():
    return lax.iota(jnp.int32, 16)


def _perm(v, idx):
    return v.at[idx].get(mode="promise_in_bounds")


def _allmax(v):
    idx = _iota16()
    for d in (1, 2, 4, 8):
        v = jnp.maximum(v, _perm(v, idx ^ d))
    return v


def _allsum(v):
    idx = _iota16()
    for d in (1, 2, 4, 8):
        v = v + _perm(v, idx ^ d)
    return v


def _mi(m):
    return jnp.where(m, jnp.full((16,), 1, jnp.int32),
                     jnp.zeros((16,), jnp.int32))


def _prefix_sum_incl(x):
    idx = _iota16()
    for d in (1, 2, 4, 8):
        sh = _perm(x, jnp.maximum(idx - d, 0))
        x = x + jnp.where(idx >= d, sh, jnp.zeros((16,), x.dtype))
    return x


def _cmpex(k, v, j, wm):
    idx = _iota16()
    pidx = idx ^ j
    pk = _perm(k, pidx)
    pv = _perm(v, pidx)
    mx = jnp.maximum(k, pk)
    mn = jnp.minimum(k, pk)
    nk = jnp.where(wm > 0, mx, mn)
    took = nk != k
    return nk, jnp.where(took, pv, v)


def _log2(j):
    return j.bit_length() - 1


def _sort16_desc(k, v):
    idx = _iota16()
    for size in (2, 4, 8, 16):
        bit = lax.shift_right_logical(idx, _log2(size)) & 1
        j = size // 2
        while j >= 1:
            upper = lax.shift_right_logical(idx, _log2(j)) & 1
            k, v = _cmpex(k, v, j, (1 - bit) ^ upper)
            j //= 2
    return k, v


def _bmerge16_desc(k, v):
    idx = _iota16()
    for j in (8, 4, 2, 1):
        upper = lax.shift_right_logical(idx, _log2(j)) & 1
        k, v = _cmpex(k, v, j, 1 - upper)
    return k, v


def _merge16_desc(ak, av, bk, bv):
    rbk = jnp.flip(bk, 0)
    rbv = jnp.flip(bv, 0)
    m = ak >= rbk
    hk = jnp.where(m, ak, rbk)
    hv = jnp.where(m, av, rbv)
    lk = jnp.where(m, rbk, ak)
    lv = jnp.where(m, rbv, av)
    hk, hv = _bmerge16_desc(hk, hv)
    lk, lv = _bmerge16_desc(lk, lv)
    return hk, hv, lk, lv


def _cross(ak, av, bk, bv):
    m = ak >= bk
    return (jnp.where(m, ak, bk), jnp.where(m, av, bv),
            jnp.where(m, bk, ak), jnp.where(m, bv, av))


def _top32_of_64_desc(keys, vals):
    s = [_sort16_desc(keys[j], vals[j]) for j in range(4)]
    a0k, a0v, a1k, a1v = _merge16_desc(s[0][0], s[0][1], s[1][0], s[1][1])
    b0k, b0v, b1k, b1v = _merge16_desc(s[2][0], s[2][1], s[3][0], s[3][1])
    rb0k, rb0v = jnp.flip(b1k, 0), jnp.flip(b1v, 0)
    rb1k, rb1v = jnp.flip(b0k, 0), jnp.flip(b0v, 0)
    h0k, h0v, _, _ = _cross(a0k, a0v, rb0k, rb0v)
    h1k, h1v, _, _ = _cross(a1k, a1v, rb1k, rb1v)
    h0k, h0v, h1k, h1v = _cross(h0k, h0v, h1k, h1v)
    h0k, h0v = _bmerge16_desc(h0k, h0v)
    h1k, h1v = _bmerge16_desc(h1k, h1v)
    return (h0k, h1k), (h0v, h1v)


def _sc_select_attend(scores3d, rowmax, memory_values, eids, mmask):
    M = rowmax.shape[0]
    COLS = scores3d.shape[1] * scores3d.shape[2]
    VD = memory_values.shape[1]
    QPW = M // _NW
    mesh = plsc.VectorSubcoreMesh(core_axis_name="c", subcore_axis_name="s")

    @functools.partial(
        pl.kernel,
        mesh=mesh,
        out_type=[
            jax.ShapeDtypeStruct((M, K_TOP), jnp.float32),
            jax.ShapeDtypeStruct((M, K_TOP), jnp.int32),
            jax.ShapeDtypeStruct((M, VD), jnp.float32),
        ],
        scratch_types=[
            pltpu.VMEM((ROWS,), jnp.float32),
            pltpu.VMEM((64,), jnp.int32),
            pltpu.VMEM((_K_ROWS, 8, COLS // 8), jnp.float32),
            pltpu.VMEM((_CAND,), jnp.float32),
            pltpu.VMEM((_CAND,), jnp.int32),
            pltpu.VMEM((K_TOP,), jnp.int32),
            pltpu.VMEM((K_TOP, VD), jnp.float32),
            pltpu.VMEM((K_TOP,), jnp.int32),
            pltpu.VMEM((K_TOP,), jnp.int32),
            pltpu.VMEM((K_TOP,), jnp.float32),
            pltpu.VMEM((VD,), jnp.float32),
            pltpu.VMEM((16,), jnp.int32),
            pltpu.SemaphoreType.DMA,
            pltpu.SemaphoreType.DMA,
            pltpu.SemaphoreType.DMA,
        ],
    )
    def k(scores_hbm, rowmax_hbm, values_hbm, eids_hbm, mmask_hbm,
          attw_hbm, eout_hbm, ret_hbm,
          rm_v, rid_v, grows_v, cv_v, ci_v, tid_v, vals_v, eid_v,
          msk_v, w_v, ret_v, cb_v, sem0, sem1, sem2):
        wid = lax.axis_index("s") * _NC + lax.axis_index("c")

        for j in range(4):
            rid_v[pl.ds(16 * j, 16)] = jnp.zeros((16,), jnp.int32)

        def per_query(it, _):
            q = wid * QPW + it

            pltpu.sync_copy(rowmax_hbm.at[q], rm_v)
            rms = [rm_v[pl.ds(16 * j, 16)] for j in range(ROWS // 16)]
            mn = rms[0]
            mx = rms[0]
            for v in rms[1:]:
                mn = jnp.minimum(mn, v)
                mx = jnp.maximum(mx, v)
            lo = -_s(_allmax(-mn))
            hi = _s(_allmax(mx))
            for _bs in range(28):
                mid = 0.5 * (lo + hi)
                cnt = jnp.zeros((16,), jnp.int32)
                for v in rms:
                    cnt = cnt + _mi(v >= mid)
                ge = _s(_allsum(cnt)) >= K_TOP
                lo = jnp.where(ge, mid, lo)
                hi = jnp.where(ge, hi, mid)
            t = lo

            base = jnp.zeros((), jnp.int32)
            for j in range(ROWS // 16):
                m = rms[j] >= t
                mi = _mi(m)
                pfx = _prefix_sum_incl(mi) - mi
                key = jnp.where(m, 1000 - pfx, jnp.full((16,), -1, jnp.int32))
                _, sperm = _sort16_desc(key, _iota16())
                gid = q * ROWS + 16 * j + sperm
                rid_v[pl.ds(base, 16)] = gid
                base = jnp.minimum(base + _s(_allsum(mi)),
                                   jnp.full((), _K_ROWS, jnp.int32))
            count = base

            pltpu.async_copy(
                scores_hbm.at[rid_v.at[pl.ds(0, _K_ROWS)]], grows_v, sem0
            ).wait()

            for j in range(_CAND // 16):
                cv_v[pl.ds(16 * j, 16)] = jnp.full((16,), _NEG, jnp.float32)
                ci_v[pl.ds(16 * j, 16)] = jnp.zeros((16,), jnp.int32)
            cb_v[...] = jnp.zeros((16,), jnp.int32)

            idx = _iota16()

            def per_row(r, _):
                rvalid = r < count
                for c0 in range(COLS // 128):
                    vs = [grows_v[r, c0, pl.ds(16 * u, 16)]
                          for u in range(8)]
                    gmax = vs[0]
                    for u in range(1, 8):
                        gmax = jnp.maximum(gmax, vs[u])
                    vmax = _s(_allmax(gmax))
                    hit = (vmax >= t) & rvalid

                    def detail():
                        cntv = jnp.zeros((16,), jnp.int32)
                        for u in range(8):
                            cntv = cntv + _mi(vs[u] >= t)
                        cnt = _s(_allsum(cntv))
                        vcur = list(vs)
                        vmax0 = vmax
                        for rnd in range(3):
                            if rnd == 0:
                                vm = vmax0
                            else:
                                gm = vcur[0]
                                for u in range(1, 8):
                                    gm = jnp.maximum(gm, vcur[u])
                                vm = _s(_allmax(gm))
                            posv = jnp.full((16,), 999, jnp.int32)
                            for u in range(8):
                                posv = jnp.minimum(
                                    posv,
                                    jnp.where(vcur[u] == vm, 16 * u + idx,
                                              jnp.full((16,), 999, jnp.int32)))
                            pos = -_s(_allmax(-posv))
                            enc1 = r * COLS + c0 * 128 + pos
                            cb = _s(cb_v[...])
                            cv_v[pl.ds(cb, 16)] = jnp.where(
                                idx == 0, jnp.full((16,), vm, jnp.float32),
                                jnp.full((16,), _NEG, jnp.float32))
                            ci_v[pl.ds(cb, 16)] = jnp.where(
                                idx == 0, jnp.full((16,), enc1, jnp.int32),
                                jnp.zeros((16,), jnp.int32))
                            cb_v[...] = jnp.minimum(
                                cb_v[...] + jnp.where(cnt > rnd, 1, 0),
                                jnp.full((16,), _CAND - 16, jnp.int32))
                            if rnd < 2:
                                vnext = []
                                for u in range(8):
                                    vnext.append(jnp.where(
                                        (16 * u + idx) == pos,
                                        jnp.full((16,), _NEG, jnp.float32),
                                        vcur[u]))
                                vcur = vnext

                    lax.cond(hit, detail, lambda: None)
                return 0

            lax.fori_loop(0, _K_ROWS, per_row, 0)


            ck = [cv_v[pl.ds(16 * j, 16)] for j in range(4)]
            cid = [ci_v[pl.ds(16 * j, 16)] for j in range(4)]
            (tk0, tk1), (tid0, tid1) = _top32_of_64_desc(ck, cid)

            rid0 = rid_v[pl.ds(0, 16)]
            rid1 = rid_v[pl.ds(16, 16)]
            rid2 = rid_v[pl.ds(32, 16)]

            def remap(enc):
                rslot = lax.shift_right_logical(enc, 10)
                col = enc & (COLS - 1)
                low = rslot & 15
                g0 = _perm(rid0, low)
                g1 = _perm(rid1, low)
                g2 = _perm(rid2, low)
                gro = jnp.where(rslot < 16, g0, jnp.where(rslot < 32, g1, g2))
                return (gro - q * ROWS) * COLS + col

            tid_v[pl.ds(0, 16)] = remap(tid0)
            tid_v[pl.ds(16, 16)] = remap(tid1)

            cpv = pltpu.async_copy(values_hbm.at[tid_v], vals_v, sem0)
            cpe = pltpu.async_copy(eids_hbm.at[tid_v], eid_v, sem1)
            cpm = pltpu.async_copy(mmask_hbm.at[tid_v], msk_v, sem2)
            cpv.wait()
            cpe.wait()
            cpm.wait()
            msk0 = msk_v[pl.ds(0, 16)]
            msk1 = msk_v[pl.ds(16, 16)]

            pen0 = jnp.where(msk0 > 0, jnp.zeros((16,), jnp.float32),
                             jnp.full((16,), _LARGE_NUMBER, jnp.float32))
            pen1 = jnp.where(msk1 > 0, jnp.zeros((16,), jnp.float32),
                             jnp.full((16,), _LARGE_NUMBER, jnp.float32))
            l0 = tk0 - pen0
            l1 = tk1 - pen1
            mxl = _s(_allmax(jnp.maximum(l0, l1)))
            e0 = jnp.exp(l0 - mxl)
            e1 = jnp.exp(l1 - mxl)
            ssum = _s(_allsum(e0 + e1))
            w0 = e0 / ssum
            w1 = e1 / ssum
            w_v[pl.ds(0, 16)] = w0
            w_v[pl.ds(16, 16)] = w1

            wks = [_perm(w0 if kk < 16 else w1,
                         jnp.full((16,), kk % 16, jnp.int32))
                   for kk in range(K_TOP)]

            def per_chunk(jc, _):
                acc = jnp.zeros((16,), jnp.float32)
                for kk in range(K_TOP):
                    acc = acc + wks[kk] * vals_v[kk, pl.ds(jc * 16, 16)]
                ret_v[pl.ds(jc * 16, 16)] = acc
                return 0

            lax.fori_loop(0, VD // 16, per_chunk, 0)

            pltpu.sync_copy(w_v, attw_hbm.at[q])
            pltpu.sync_copy(eid_v, eout_hbm.at[q])
            pltpu.sync_copy(ret_v, ret_hbm.at[q])
            return 0

        lax.fori_loop(0, QPW, per_query, 0)

    return k(scores3d, rowmax, memory_values, eids, mmask)


def _finish_body(enc_ref, ret_ref, wv_ref, bv_ref, mm_ref, pos_ref,
                 scale_ref, bias_ref, out_ref, *, tpb):
    i = pl.program_id(0)
    upd = (
        jnp.dot(ret_ref[...], wv_ref[...], preferred_element_type=jnp.float32)
        + bv_ref[...]
    ) * mm_ref[...].reshape(-1, 1).astype(jnp.float32)
    pos = pos_ref[...].reshape(-1)
    tok = lax.broadcasted_iota(jnp.int32, (pos.shape[0], tpb), 1) + i * tpb
    oh = (pos[:, None] == tok).astype(jnp.float32)
    scat = lax.dot_general(
        oh, upd, (((0,), (0,)), ((), ())),
        preferred_element_type=jnp.float32,
    )
    enc = enc_ref[...] + scat
    mean = jnp.mean(enc, axis=-1, keepdims=True)
    var = jnp.mean((enc - mean) ** 2, axis=-1, keepdims=True)
    out_ref[...] = (enc - mean) * lax.rsqrt(var + LN_EPS) * scale_ref[...] + bias_ref[...]


def _finish(encoding_flat, retrieved, w_value, b_value, mention_mask, pos,
            ln_scale, ln_bias):
    N, H = encoding_flat.shape
    M, VD = retrieved.shape
    TPB = 256
    grid = N // TPB
    return pl.pallas_call(
        functools.partial(_finish_body, tpb=TPB),
        grid=(grid,),
        in_specs=[
            pl.BlockSpec((TPB, H), lambda i: (i, 0)),
            pl.BlockSpec((M, VD), lambda i: (0, 0)),
            pl.BlockSpec((VD, H), lambda i: (0, 0)),
            pl.BlockSpec((1, H), lambda i: (0, 0)),
            pl.BlockSpec((M, 1), lambda i: (0, 0)),
            pl.BlockSpec((1, M), lambda i: (0, 0)),
            pl.BlockSpec((1, H), lambda i: (0, 0)),
            pl.BlockSpec((1, H), lambda i: (0, 0)),
        ],
        out_specs=pl.BlockSpec((TPB, H), lambda i: (i, 0)),
        out_shape=jax.ShapeDtypeStruct((N, H), jnp.float32),
    )(encoding_flat, retrieved, w_value, b_value.reshape(1, H),
      mention_mask.reshape(M, 1), pos.reshape(1, M), ln_scale.reshape(1, H),
      ln_bias.reshape(1, H))


def kernel(encoding, mention_batch_positions, mention_start_positions,
           mention_end_positions, mention_mask, memory_keys, memory_values,
           memory_mask, memory_entity_ids, w_query, b_query, w_value,
           b_value, ln_scale, ln_bias, deterministic=True):
    B, T, H = encoding.shape
    M = mention_batch_positions.shape[0]
    MEM, KD = memory_keys.shape
    cols = MEM // ROWS
    flat = encoding.reshape(B * T, H)

    start_pos = (mention_batch_positions * T + mention_start_positions).astype(jnp.int32)
    end_pos = (mention_batch_positions * T + mention_end_positions).astype(jnp.int32)
    qin = jnp.concatenate((flat[start_pos], flat[end_pos]), axis=-1)
    queries = _compute_queries(qin, w_query, b_query)

    scores, rowmax3 = _compute_scores(queries, memory_keys)
    rowmax = jnp.transpose(rowmax3, (1, 0, 2)).reshape(M, ROWS)

    attention_weights, top_entity_ids, retrieved = _sc_select_attend(
        scores.reshape(M * ROWS, 8, cols // 8), rowmax, memory_values,
        memory_entity_ids.astype(jnp.int32), memory_mask.astype(jnp.int32))

    normed = _finish(flat, retrieved, w_value, b_value, mention_mask,
                     start_pos, ln_scale, ln_bias).reshape(B, T, H)
    return (normed, attention_weights, top_entity_ids)
